# baseline (speedup 1.0000x reference)
import functools

import jax
import jax.numpy as jnp
from jax import lax
from jax.experimental import pallas as pl
from jax.experimental.pallas import tpu as pltpu
from jax.experimental.pallas import tpu_sc as plsc

_NPASS = 1


def _transform_stage(features, weight):
    n, nin = features.shape
    fv, _, nout = weight.shape

    def body(x_ref, w_ref, o_ref):
        o_ref[...] = jnp.dot(
            x_ref[...], w_ref[0], preferred_element_type=jnp.float32
        ).astype(jnp.bfloat16)

    return pl.pallas_call(
        body,
        grid=(fv,),
        in_specs=[
            pl.BlockSpec((n, nin), lambda f: (0, 0)),
            pl.BlockSpec((1, nin, nout), lambda f: (f, 0, 0)),
        ],
        out_specs=pl.BlockSpec((n, nout), lambda f: (f, 0)),
        out_shape=jax.ShapeDtypeStruct((fv * n, nout), jnp.bfloat16),
    )(features, weight)


def _combine_stage(partials, bias):
    nc, n, nout = partials.shape

    def body(p_ref, b_ref, o_ref):
        o_ref[...] = (
            jnp.sum(p_ref[...].astype(jnp.float32), axis=0) + b_ref[...]
        )

    return pl.pallas_call(
        body,
        in_specs=[
            pl.BlockSpec((nc, n, nout), lambda: (0, 0, 0)),
            pl.BlockSpec((1, nout), lambda: (0, 0)),
        ],
        out_specs=pl.BlockSpec((n, nout), lambda: (0, 0)),
        out_shape=jax.ShapeDtypeStruct((n, nout), jnp.float32),
    )(partials, bias.reshape(1, nout))


def _scatter_stage(transformed, gidx, dst, nacc, nc, ns, chunk, cpt):
    nout = transformed.shape[1]
    rpt = nacc // ns
    lanes = nout // 32
    cpp = cpt // _NPASS
    mesh = plsc.VectorSubcoreMesh(core_axis_name="c", subcore_axis_name="s")

    @functools.partial(
        pl.kernel,
        mesh=mesh,
        out_type=jax.ShapeDtypeStruct((nc, nacc, nout), jnp.bfloat16),
        scratch_types=[
            pltpu.VMEM((cpp, chunk), jnp.int32),
            pltpu.VMEM((cpp, chunk), jnp.int32),
            pltpu.VMEM((chunk, nout), jnp.bfloat16),
            pltpu.VMEM((chunk, nout), jnp.bfloat16),
            pltpu.VMEM_SHARED((nacc, nout), jnp.bfloat16),
            pltpu.SemaphoreType.DMA,
            pltpu.SemaphoreType.DMA,
        ],
        compiler_params=pltpu.CompilerParams(use_tc_tiling_on_sc=False),
    )
    def sc_fn(tr_hbm, gidx_hbm, dst_hbm, part_hbm, gall, dall, r0, r1,
              acc, sem0, sem1):
        cid = lax.axis_index("c")
        sid = lax.axis_index("s")
        wid = cid * ns + sid

        zvec = jnp.zeros((32,), jnp.bfloat16)

        def zbody(i, _):
            r0[i // lanes, pl.ds((i % lanes) * 32, 32)] = zvec
            return 0

        lax.fori_loop(0, chunk * lanes, zbody, 0)
        for q in range(rpt // chunk):
            pltpu.sync_copy(r0, acc.at[pl.ds(sid * rpt + q * chunk, chunk)])
        plsc.subcore_barrier()

        def gather_start(j, rbuf, sem):
            pltpu.async_copy(tr_hbm.at[gall.at[j]], rbuf, sem)

        def gather_wait(j, rbuf, sem):
            pltpu.make_async_copy(tr_hbm.at[gall.at[j]], rbuf, sem).wait()

        def scatter_add(j, rbuf):
            pltpu.sync_copy(rbuf, acc.at[dall.at[j]], add=True)

        for p in range(_NPASS):
            base = wid * cpt + p * cpp
            pltpu.sync_copy(gidx_hbm.at[pl.ds(base, cpp)], gall)
            pltpu.sync_copy(dst_hbm.at[pl.ds(base, cpp)], dall)
            gather_start(0, r0, sem0)

            def body(t, _):
                j0 = 2 * t
                j1 = 2 * t + 1
                j2 = 2 * t + 2
                gather_wait(j0, r0, sem0)
                gather_start(j1, r1, sem1)
                scatter_add(j0, r0)
                gather_wait(j1, r1, sem1)

                @pl.when(j2 < cpp)
                def _():
                    gather_start(j2, r0, sem0)

                scatter_add(j1, r1)
                return 0

            lax.fori_loop(0, cpp // 2, body, 0)

        plsc.subcore_barrier()

        pltpu.sync_copy(
            acc.at[pl.ds(sid * rpt, rpt)],
            part_hbm.at[cid, pl.ds(sid * rpt, rpt)],
        )

    return sc_fn(transformed, gidx, dst)


def kernel(features, weight, bias, edge_index, offset_id):
    n, nin = features.shape
    fv, _, nout = weight.shape
    e = edge_index.shape[1]

    info = plsc.get_sparse_core_info()
    nc, ns = info.num_cores, info.num_subcores
    nw = nc * ns

    chunk = 128
    cpt = -(-e // (chunk * nw))
    cpt = -(-cpt // (2 * _NPASS)) * (2 * _NPASS)
    ep = cpt * chunk * nw

    rpt = -(-(n + 1) // (ns * chunk)) * chunk
    nacc = rpt * ns

    src = edge_index[0].astype(jnp.int32)
    dst = edge_index[1].astype(jnp.int32)
    off = offset_id.astype(jnp.int32)
    gidx = off * n + src
    pad = ep - e
    gidx_p = jnp.concatenate([gidx, jnp.zeros((pad,), jnp.int32)])
    dst_p = jnp.concatenate([dst, jnp.full((pad,), n, jnp.int32)])

    transformed = _transform_stage(features, weight)
    partials = _scatter_stage(
        transformed,
        gidx_p.reshape(ep // chunk, chunk),
        dst_p.reshape(ep // chunk, chunk),
        nacc,
        nc,
        ns,
        chunk,
        cpt,
    )
    return _combine_stage(partials[:, :n], bias)


# speedup vs baseline: 1.0381x; 1.0381x over previous
import functools

import jax
import jax.numpy as jnp
from jax import lax
from jax.experimental import pallas as pl
from jax.experimental.pallas import tpu as pltpu
from jax.experimental.pallas import tpu_sc as plsc

_NPASS = 1


def _transform_stage(features, weight):
    n, nin = features.shape
    fv, _, nout = weight.shape

    def body(x_ref, w_ref, o_ref):
        o_ref[...] = jnp.dot(
            x_ref[...], w_ref[0], preferred_element_type=jnp.float32
        ).astype(jnp.bfloat16)

    return pl.pallas_call(
        body,
        grid=(fv,),
        in_specs=[
            pl.BlockSpec((n, nin), lambda f: (0, 0)),
            pl.BlockSpec((1, nin, nout), lambda f: (f, 0, 0)),
        ],
        out_specs=pl.BlockSpec((n, nout), lambda f: (f, 0)),
        out_shape=jax.ShapeDtypeStruct((fv * n, nout), jnp.bfloat16),
    )(features, weight)


def _combine_stage(partials, bias):
    nc, n, nout = partials.shape

    def body(p_ref, b_ref, o_ref):
        o_ref[...] = (
            jnp.sum(p_ref[...].astype(jnp.float32), axis=0) + b_ref[...]
        )

    return pl.pallas_call(
        body,
        in_specs=[
            pl.BlockSpec((nc, n, nout), lambda: (0, 0, 0)),
            pl.BlockSpec((1, nout), lambda: (0, 0)),
        ],
        out_specs=pl.BlockSpec((n, nout), lambda: (0, 0)),
        out_shape=jax.ShapeDtypeStruct((n, nout), jnp.float32),
    )(partials, bias.reshape(1, nout))


def _scatter_stage(transformed, gidx, dst, nacc, nc, ns, chunk, cpt):
    nout = transformed.shape[1]
    rpt = nacc // ns
    lanes = nout // 32
    cpp = cpt // _NPASS
    mesh = plsc.VectorSubcoreMesh(core_axis_name="c", subcore_axis_name="s")

    nbuf = 8

    @functools.partial(
        pl.kernel,
        mesh=mesh,
        out_type=jax.ShapeDtypeStruct((nc, nacc, nout), jnp.bfloat16),
        scratch_types=[
            pltpu.VMEM((cpp, chunk), jnp.int32),
            pltpu.VMEM((cpp, chunk), jnp.int32),
            [pltpu.VMEM((chunk, nout), jnp.bfloat16) for _ in range(nbuf)],
            pltpu.VMEM_SHARED((nacc, nout), jnp.bfloat16),
            [pltpu.SemaphoreType.DMA for _ in range(nbuf)],
            [pltpu.SemaphoreType.DMA for _ in range(nbuf)],
        ],
        compiler_params=pltpu.CompilerParams(use_tc_tiling_on_sc=False),
    )
    def sc_fn(tr_hbm, gidx_hbm, dst_hbm, part_hbm, gall, dall, rbufs,
              acc, sg, ss):
        cid = lax.axis_index("c")
        sid = lax.axis_index("s")
        wid = cid * ns + sid

        zvec = jnp.zeros((32,), jnp.bfloat16)

        def zbody(i, _):
            rbufs[0][i // lanes, pl.ds((i % lanes) * 32, 32)] = zvec
            return 0

        lax.fori_loop(0, chunk * lanes, zbody, 0)
        for q in range(rpt // chunk):
            pltpu.sync_copy(
                rbufs[0], acc.at[pl.ds(sid * rpt + q * chunk, chunk)]
            )
        plsc.subcore_barrier()

        def gather_start(b, j):
            pltpu.async_copy(tr_hbm.at[gall.at[j]], rbufs[b], sg[b])

        def gather_wait(b, j):
            pltpu.make_async_copy(tr_hbm.at[gall.at[j]], rbufs[b], sg[b]).wait()

        def scatter_start(b, j):
            pltpu.async_copy(rbufs[b], acc.at[dall.at[j]], ss[b], add=True)

        def scatter_wait(b, j):
            pltpu.make_async_copy(rbufs[b], acc.at[dall.at[j]], ss[b]).wait()

        base = wid * cpt
        pltpu.sync_copy(gidx_hbm.at[pl.ds(base, cpp)], gall)
        pltpu.sync_copy(dst_hbm.at[pl.ds(base, cpp)], dall)
        for b in range(nbuf):
            gather_start(b, b)

        def body(t, _):
            for b in range(nbuf):
                j = t * nbuf + b
                gather_wait(b, j)
                scatter_start(b, j)
                b2 = (b + 6) % nbuf
                jn = j + 6

                @pl.when(jnp.logical_and(j >= 2, jn < cpp))
                def _():
                    scatter_wait(b2, j - 2)
                    gather_start(b2, jn)

            return 0

        lax.fori_loop(0, cpp // nbuf, body, 0)
        for b in range(nbuf):
            scatter_wait(b, 0)

        plsc.subcore_barrier()

        pltpu.sync_copy(
            acc.at[pl.ds(sid * rpt, rpt)],
            part_hbm.at[cid, pl.ds(sid * rpt, rpt)],
        )

    return sc_fn(transformed, gidx, dst)


def kernel(features, weight, bias, edge_index, offset_id):
    n, nin = features.shape
    fv, _, nout = weight.shape
    e = edge_index.shape[1]

    info = plsc.get_sparse_core_info()
    nc, ns = info.num_cores, info.num_subcores
    nw = nc * ns

    chunk = 128
    cpt = -(-e // (chunk * nw))
    cpt = -(-cpt // 8) * 8
    ep = cpt * chunk * nw

    rpt = -(-(n + 1) // (ns * chunk)) * chunk
    nacc = rpt * ns

    src = edge_index[0].astype(jnp.int32)
    dst = edge_index[1].astype(jnp.int32)
    off = offset_id.astype(jnp.int32)
    gidx = off * n + src
    pad = ep - e
    gidx_p = jnp.concatenate([gidx, jnp.zeros((pad,), jnp.int32)])
    dst_p = jnp.concatenate([dst, jnp.full((pad,), n, jnp.int32)])

    transformed = _transform_stage(features, weight)
    partials = _scatter_stage(
        transformed,
        gidx_p.reshape(ep // chunk, chunk),
        dst_p.reshape(ep // chunk, chunk),
        nacc,
        nc,
        ns,
        chunk,
        cpt,
    )
    return _combine_stage(partials[:, :n], bias)


# speedup vs baseline: 1.0408x; 1.0026x over previous
import functools

import jax
import jax.numpy as jnp
from jax import lax
from jax.experimental import pallas as pl
from jax.experimental.pallas import tpu as pltpu
from jax.experimental.pallas import tpu_sc as plsc

_NPASS = 1


def _transform_stage(features, weight):
    n, nin = features.shape
    fv, _, nout = weight.shape

    def body(x_ref, w_ref, o_ref):
        o_ref[...] = jnp.dot(
            x_ref[...], w_ref[0], preferred_element_type=jnp.float32
        ).astype(jnp.bfloat16)

    return pl.pallas_call(
        body,
        grid=(fv,),
        in_specs=[
            pl.BlockSpec((n, nin), lambda f: (0, 0)),
            pl.BlockSpec((1, nin, nout), lambda f: (f, 0, 0)),
        ],
        out_specs=pl.BlockSpec((n, nout), lambda f: (f, 0)),
        out_shape=jax.ShapeDtypeStruct((fv * n, nout), jnp.bfloat16),
    )(features, weight)


def _combine_stage(partials, bias):
    nc, n, nout = partials.shape

    def body(p_ref, b_ref, o_ref):
        o_ref[...] = (
            jnp.sum(p_ref[...].astype(jnp.float32), axis=0) + b_ref[...]
        )

    return pl.pallas_call(
        body,
        in_specs=[
            pl.BlockSpec((nc, n, nout), lambda: (0, 0, 0)),
            pl.BlockSpec((1, nout), lambda: (0, 0)),
        ],
        out_specs=pl.BlockSpec((n, nout), lambda: (0, 0)),
        out_shape=jax.ShapeDtypeStruct((n, nout), jnp.float32),
    )(partials, bias.reshape(1, nout))


def _scatter_stage(transformed, gidx, dst, nacc, nc, ns, chunk, cpt):
    nout = transformed.shape[1]
    rpt = nacc // ns
    lanes = nout // 32
    cpp = cpt
    mesh = plsc.VectorSubcoreMesh(core_axis_name="c", subcore_axis_name="s")

    nbuf = 8

    @functools.partial(
        pl.kernel,
        mesh=mesh,
        out_type=jax.ShapeDtypeStruct((nc, nacc, nout), jnp.bfloat16),
        scratch_types=[
            pltpu.VMEM((cpp, chunk), jnp.int32),
            pltpu.VMEM((cpp, chunk), jnp.int32),
            [pltpu.VMEM((chunk, nout), jnp.bfloat16) for _ in range(nbuf)],
            pltpu.VMEM_SHARED((nacc, nout), jnp.bfloat16),
            [pltpu.SemaphoreType.DMA for _ in range(nbuf)],
            [pltpu.SemaphoreType.DMA for _ in range(nbuf)],
        ],
        compiler_params=pltpu.CompilerParams(use_tc_tiling_on_sc=False),
    )
    def sc_fn(tr_hbm, gidx_hbm, dst_hbm, part_hbm, gall, dall, rbufs,
              acc, sg, ss):
        cid = lax.axis_index("c")
        sid = lax.axis_index("s")
        wid = cid * ns + sid
        base = wid * cpt

        idx_g = pltpu.async_copy(gidx_hbm.at[pl.ds(base, cpp)], gall, ss[0])
        idx_d = pltpu.async_copy(dst_hbm.at[pl.ds(base, cpp)], dall, ss[1])

        zvec = jnp.zeros((32,), jnp.bfloat16)

        def zbody(i, _):
            rbufs[0][i // lanes, pl.ds((i % lanes) * 32, 32)] = zvec
            return 0

        lax.fori_loop(0, chunk * lanes, zbody, 0)
        for q in range(rpt // chunk):
            pltpu.sync_copy(
                rbufs[0], acc.at[pl.ds(sid * rpt + q * chunk, chunk)]
            )
        plsc.subcore_barrier()

        def gather_start(b, j):
            pltpu.async_copy(tr_hbm.at[gall.at[j]], rbufs[b], sg[b])

        def gather_wait(b, j):
            pltpu.make_async_copy(tr_hbm.at[gall.at[j]], rbufs[b], sg[b]).wait()

        def scatter_start(b, j):
            pltpu.async_copy(rbufs[b], acc.at[dall.at[j]], ss[b], add=True)

        def scatter_wait(b, j):
            pltpu.make_async_copy(rbufs[b], acc.at[dall.at[j]], ss[b]).wait()

        idx_g.wait()
        idx_d.wait()
        for b in range(nbuf):
            gather_start(b, b)

        def body(t, _):
            for b in range(nbuf):
                j = t * nbuf + b
                gather_wait(b, j)
                scatter_start(b, j)
                b2 = (b + 6) % nbuf
                jn = j + 6

                @pl.when(jnp.logical_and(j >= 2, jn < cpp))
                def _():
                    scatter_wait(b2, j - 2)
                    gather_start(b2, jn)

            return 0

        lax.fori_loop(0, cpp // nbuf, body, 0)
        for b in range(nbuf):
            scatter_wait(b, 0)

        plsc.subcore_barrier()

        pltpu.sync_copy(
            acc.at[pl.ds(sid * rpt, rpt)],
            part_hbm.at[cid, pl.ds(sid * rpt, rpt)],
        )

    return sc_fn(transformed, gidx, dst)


def kernel(features, weight, bias, edge_index, offset_id):
    n, nin = features.shape
    fv, _, nout = weight.shape
    e = edge_index.shape[1]

    info = plsc.get_sparse_core_info()
    nc, ns = info.num_cores, info.num_subcores
    nw = nc * ns

    chunk = 128
    cpt = -(-e // (chunk * nw))
    cpt = -(-cpt // 8) * 8
    ep = cpt * chunk * nw

    rpt = -(-(n + 1) // (ns * chunk)) * chunk
    nacc = rpt * ns

    src = edge_index[0].astype(jnp.int32)
    dst = edge_index[1].astype(jnp.int32)
    off = offset_id.astype(jnp.int32)
    gidx = off * n + src
    pad = ep - e
    gidx_p = jnp.concatenate([gidx, jnp.zeros((pad,), jnp.int32)])
    dst_p = jnp.concatenate([dst, jnp.full((pad,), n, jnp.int32)])

    transformed = _transform_stage(features, weight)
    partials = _scatter_stage(
        transformed,
        gidx_p.reshape(ep // chunk, chunk),
        dst_p.reshape(ep // chunk, chunk),
        nacc,
        nc,
        ns,
        chunk,
        cpt,
    )
    return _combine_stage(partials[:, :n], bias)


# speedup vs baseline: 1.0414x; 1.0005x over previous
import functools

import jax
import jax.numpy as jnp
from jax import lax
from jax.experimental import pallas as pl
from jax.experimental.pallas import tpu as pltpu
from jax.experimental.pallas import tpu_sc as plsc


def _transform_stage(features, weight):
    n, nin = features.shape
    fv, _, nout = weight.shape

    def body(x_ref, w_ref, o_ref):
        o_ref[...] = jnp.dot(
            x_ref[...], w_ref[0], preferred_element_type=jnp.float32
        ).astype(jnp.bfloat16)

    return pl.pallas_call(
        body,
        grid=(fv,),
        in_specs=[
            pl.BlockSpec((n, nin), lambda f: (0, 0)),
            pl.BlockSpec((1, nin, nout), lambda f: (f, 0, 0)),
        ],
        out_specs=pl.BlockSpec((n, nout), lambda f: (f, 0)),
        out_shape=jax.ShapeDtypeStruct((fv * n, nout), jnp.bfloat16),
    )(features, weight)


def _combine_stage(partials, bias):
    nc, n, nout = partials.shape

    def body(p_ref, b_ref, o_ref):
        o_ref[...] = (
            jnp.sum(p_ref[...].astype(jnp.float32), axis=0) + b_ref[...]
        )

    return pl.pallas_call(
        body,
        in_specs=[
            pl.BlockSpec((nc, n, nout), lambda: (0, 0, 0)),
            pl.BlockSpec((1, nout), lambda: (0, 0)),
        ],
        out_specs=pl.BlockSpec((n, nout), lambda: (0, 0)),
        out_shape=jax.ShapeDtypeStruct((n, nout), jnp.float32),
    )(partials, bias.reshape(1, nout))


def _scatter_stage(transformed, gidx, dst, nacc, nc, ns, chunk, cpt):
    nout = transformed.shape[1]
    rpt = nacc // ns
    lanes = nout // 32
    cpp = cpt
    mesh = plsc.VectorSubcoreMesh(core_axis_name="c", subcore_axis_name="s")

    nbuf = 8

    @functools.partial(
        pl.kernel,
        mesh=mesh,
        out_type=jax.ShapeDtypeStruct((nc, nacc, nout), jnp.bfloat16),
        scratch_types=[
            pltpu.VMEM((cpp, chunk), jnp.int32),
            pltpu.VMEM((cpp, chunk), jnp.int32),
            [pltpu.VMEM((chunk, nout), jnp.bfloat16) for _ in range(nbuf)],
            pltpu.VMEM_SHARED((nacc, nout), jnp.bfloat16),
            [pltpu.SemaphoreType.DMA for _ in range(nbuf)],
            [pltpu.SemaphoreType.DMA for _ in range(nbuf)],
        ],
        compiler_params=pltpu.CompilerParams(use_tc_tiling_on_sc=False),
    )
    def sc_fn(tr_hbm, gidx_hbm, dst_hbm, part_hbm, gall, dall, rbufs,
              acc, sg, ss):
        cid = lax.axis_index("c")
        sid = lax.axis_index("s")
        wid = cid * ns + sid
        base = wid * cpt

        idx_g = pltpu.async_copy(gidx_hbm.at[pl.ds(base, cpp)], gall, ss[0])
        idx_d = pltpu.async_copy(dst_hbm.at[pl.ds(base, cpp)], dall, ss[1])

        zvec = jnp.zeros((32,), jnp.bfloat16)

        def zbody(i, _):
            rbufs[0][i // lanes, pl.ds((i % lanes) * 32, 32)] = zvec
            return 0

        lax.fori_loop(0, chunk * lanes, zbody, 0)
        for q in range(rpt // chunk):
            pltpu.sync_copy(
                rbufs[0], acc.at[pl.ds(sid * rpt + q * chunk, chunk)]
            )
        plsc.subcore_barrier()

        def gather_start(b, j):
            pltpu.async_copy(tr_hbm.at[gall.at[j]], rbufs[b], sg[b])

        def gather_wait(b, j):
            pltpu.make_async_copy(tr_hbm.at[gall.at[j]], rbufs[b], sg[b]).wait()

        def scatter_start(b, j):
            pltpu.async_copy(rbufs[b], acc.at[dall.at[j]], ss[b], add=True)

        def scatter_wait(b, j):
            pltpu.make_async_copy(rbufs[b], acc.at[dall.at[j]], ss[b]).wait()

        idx_g.wait()
        idx_d.wait()
        for b in range(nbuf):
            gather_start(b, b)

        def body(t, _):
            for b in range(nbuf):
                j = t * nbuf + b
                gather_wait(b, j)
                scatter_start(b, j)
                b2 = (b + 6) % nbuf
                jn = j + 6

                @pl.when(jnp.logical_and(j >= 2, jn < cpp))
                def _():
                    scatter_wait(b2, j - 2)
                    gather_start(b2, jn)

            return 0

        lax.fori_loop(0, cpp // nbuf, body, 0)
        for b in range(nbuf):
            scatter_wait(b, 0)

        plsc.subcore_barrier()

        pltpu.sync_copy(
            acc.at[pl.ds(sid * rpt, rpt)],
            part_hbm.at[cid, pl.ds(sid * rpt, rpt)],
        )

    return sc_fn(transformed, gidx, dst)


def kernel(features, weight, bias, edge_index, offset_id):
    n, nin = features.shape
    fv, _, nout = weight.shape
    e = edge_index.shape[1]

    info = plsc.get_sparse_core_info()
    nc, ns = info.num_cores, info.num_subcores
    nw = nc * ns

    chunk = 128
    cpt = -(-e // (chunk * nw))
    cpt = -(-cpt // 8) * 8
    ep = cpt * chunk * nw

    rpt = -(-(n + 1) // (ns * chunk)) * chunk
    nacc = rpt * ns

    src = edge_index[0].astype(jnp.int32)
    dst = edge_index[1].astype(jnp.int32)
    off = offset_id.astype(jnp.int32)
    gidx = off * n + src
    pad = ep - e
    gidx_p = jnp.concatenate([gidx, jnp.zeros((pad,), jnp.int32)])
    dst_p = jnp.concatenate([dst, jnp.full((pad,), n, jnp.int32)])

    transformed = _transform_stage(features, weight)
    partials = _scatter_stage(
        transformed,
        gidx_p.reshape(ep // chunk, chunk),
        dst_p.reshape(ep // chunk, chunk),
        nacc,
        nc,
        ns,
        chunk,
        cpt,
    )
    return _combine_stage(partials[:, :n], bias)
